# baseline (speedup 1.0000x reference)
import functools

import jax
import jax.numpy as jnp
from jax import lax
from jax.experimental import pallas as pl
from jax.experimental.pallas import tpu as pltpu
from jax.experimental.pallas import tpu_sc as plsc

HIDDEN = 1024
EPS = 1e-12

_NC = 2
_NS = 16
_NW = _NC * _NS


def _make_sc_gather(n_tok: int, ch: int):
    n_per_w = n_tok // _NW
    nch = n_per_w // ch
    mesh = plsc.VectorSubcoreMesh(core_axis_name="c", subcore_axis_name="s")

    @functools.partial(
        pl.kernel,
        mesh=mesh,
        out_type=jax.ShapeDtypeStruct((n_tok, HIDDEN), jnp.float32),
        scratch_types=[
            pltpu.VMEM((n_per_w,), jnp.int32),
            pltpu.VMEM((ch, HIDDEN), jnp.float32),
            pltpu.VMEM((ch, HIDDEN), jnp.float32),
            pltpu.SemaphoreType.DMA,
            pltpu.SemaphoreType.DMA,
            pltpu.SemaphoreType.DMA,
        ],
    )
    def gather_k(ids_hbm, table_hbm, out_hbm, ids_v, r0, r1, gsem, w0sem, w1sem):
        wid = lax.axis_index("s") * _NC + lax.axis_index("c")
        base = wid * n_per_w
        pltpu.sync_copy(ids_hbm.at[pl.ds(base, n_per_w)], ids_v)

        bufs = (r0, r1)
        wsems = (w0sem, w1sem)

        def start_gather(c, buf):
            return pltpu.async_copy(
                table_hbm.at[ids_v.at[pl.ds(c * ch, ch)]], buf, gsem
            )

        pending = [None, None]
        g_prev = start_gather(0, r0)
        for c in range(nch):
            b = c % 2
            g_prev.wait()
            if c + 1 < nch:
                nb = (c + 1) % 2
                if pending[nb] is not None:
                    pending[nb].wait()
                    pending[nb] = None
                g_prev = start_gather(c + 1, bufs[nb])
            pending[b] = pltpu.async_copy(
                bufs[b], out_hbm.at[pl.ds(base + c * ch, ch)], wsems[b]
            )
        for b in range(2):
            if pending[b] is not None:
                pending[b].wait()

    return gather_k


def _make_tc_ln(n_tok: int, seq_len: int, batch: int, blk: int):
    pos_blocks = seq_len // blk

    def body(x_ref, pos_ref, type_ref, g_ref, b_ref, o_ref):
        x = x_ref[...] + pos_ref[...] + type_ref[0:1, :]
        mu = jnp.mean(x, axis=-1, keepdims=True)
        xc = x - mu
        var = jnp.mean(xc * xc, axis=-1, keepdims=True)
        rstd = lax.rsqrt(var + EPS)
        o_ref[...] = (xc * rstd) * g_ref[0:1, :] + b_ref[0:1, :]

    return pl.pallas_call(
        body,
        grid=(pos_blocks, batch),
        in_specs=[
            pl.BlockSpec((blk, HIDDEN), lambda p, b: (b * pos_blocks + p, 0)),
            pl.BlockSpec((blk, HIDDEN), lambda p, b: (p, 0)),
            pl.BlockSpec((2, HIDDEN), lambda p, b: (0, 0)),
            pl.BlockSpec((1, HIDDEN), lambda p, b: (0, 0)),
            pl.BlockSpec((1, HIDDEN), lambda p, b: (0, 0)),
        ],
        out_specs=pl.BlockSpec((blk, HIDDEN), lambda p, b: (b * pos_blocks + p, 0)),
        out_shape=jax.ShapeDtypeStruct((n_tok, HIDDEN), jnp.float32),
    )


def _make_sc_fused(batch: int, seq_len: int, ch: int):
    n_tok = batch * seq_len
    pw = seq_len // _NW
    cpb = pw // ch
    nch = batch * cpb
    vpr = HIDDEN // 16
    mesh = plsc.VectorSubcoreMesh(core_axis_name="c", subcore_axis_name="s")

    @functools.partial(
        pl.kernel,
        mesh=mesh,
        out_type=jax.ShapeDtypeStruct((n_tok, HIDDEN), jnp.float32),
        scratch_types=[
            pltpu.VMEM((batch * pw,), jnp.int32),
            pltpu.VMEM((pw, HIDDEN), jnp.float32),
            pltpu.VMEM((HIDDEN,), jnp.float32),
            pltpu.VMEM((HIDDEN,), jnp.float32),
            pltpu.VMEM((HIDDEN,), jnp.float32),
            pltpu.VMEM((ch, HIDDEN), jnp.float32),
            pltpu.VMEM((ch, HIDDEN), jnp.float32),
            pltpu.VMEM((ch, HIDDEN), jnp.float32),
            pltpu.VMEM((ch, HIDDEN), jnp.float32),
            pltpu.SemaphoreType.DMA,
            pltpu.SemaphoreType.DMA,
            pltpu.SemaphoreType.DMA,
            pltpu.SemaphoreType.DMA,
        ],
    )
    def fused_k(ids_hbm, table_hbm, pos_hbm, type_hbm, gamma_hbm, beta_hbm,
                out_hbm, ids_v, pt_v, type_v, gamma_v, beta_v,
                g0, g1, w0, w1, gs0, gs1, ws0, ws1):
        wid = lax.axis_index("s") * _NC + lax.axis_index("c")
        pbase = wid * pw
        for b in range(batch):
            pltpu.sync_copy(
                ids_hbm.at[pl.ds(b * seq_len + pbase, pw)],
                ids_v.at[pl.ds(b * pw, pw)],
            )
        pltpu.sync_copy(pos_hbm.at[pl.ds(pbase, pw)], pt_v)
        pltpu.sync_copy(type_hbm.at[0], type_v)
        pltpu.sync_copy(gamma_hbm, gamma_v)
        pltpu.sync_copy(beta_hbm, beta_v)

        def fold_type(r, carry):
            for j in range(vpr):
                js = pl.ds(j * 16, 16)
                pt_v[r, js] = pt_v[r, js] + type_v[js]
            return carry

        lax.fori_loop(0, pw, fold_type, 0)

        def gather_desc(c, gbuf, gsem):
            return pltpu.make_async_copy(
                table_hbm.at[ids_v.at[pl.ds(c * ch, ch)]], gbuf, gsem
            )

        def write_desc(c, wbuf, wsem):
            b = c // cpb
            lc = lax.rem(c, cpb)
            dst = b * seq_len + pbase + lc * ch
            return pltpu.make_async_copy(
                wbuf, out_hbm.at[pl.ds(dst, ch)], wsem
            )

        def compute_chunk(c, gbuf, wbuf):
            lc = lax.rem(c, cpb)

            def row_body(r, carry):
                s = jnp.zeros((16,), jnp.float32)
                s2 = jnp.zeros((16,), jnp.float32)
                pr = lc * ch + r
                for j in range(vpr):
                    js = pl.ds(j * 16, 16)
                    x = gbuf[r, js] + pt_v[pr, js]
                    gbuf[r, js] = x
                    s = s + x
                    s2 = s2 + x * x
                def permute(x, idx):
                    return lax.gather(
                        x,
                        idx[:, None],
                        lax.GatherDimensionNumbers(
                            offset_dims=(),
                            collapsed_slice_dims=(0,),
                            start_index_map=(0,),
                        ),
                        slice_sizes=(1,),
                        mode=lax.GatherScatterMode.PROMISE_IN_BOUNDS,
                    )

                for d in (1, 2, 4, 8):
                    idx = lax.iota(jnp.int32, 16) ^ d
                    s = s + permute(s, idx)
                    s2 = s2 + permute(s2, idx)
                tot = s
                tot2 = s2
                mean = tot * (1.0 / HIDDEN)
                var = tot2 * (1.0 / HIDDEN) - mean * mean
                v = var + EPS
                seed = lax.bitcast_convert_type(
                    jnp.int32(0x5F3759DF) - lax.shift_right_logical(
                        lax.bitcast_convert_type(v, jnp.int32), 1
                    ),
                    jnp.float32,
                )
                hv = 0.5 * v
                y = seed
                for _ in range(3):
                    y = y * (1.5 - hv * y * y)
                for j in range(vpr):
                    js = pl.ds(j * 16, 16)
                    xn = (gbuf[r, js] - mean) * y
                    wbuf[r, js] = xn * gamma_v[js] + beta_v[js]
                return carry

            lax.fori_loop(0, ch, row_body, 0)

        gather_desc(0, g0, gs0).start()
        gather_desc(1, g1, gs1).start()

        def body2(i, carry):
            for (gbuf, wbuf, gsem, wsem, par) in (
                (g0, w0, gs0, ws0, 0),
                (g1, w1, gs1, ws1, 1),
            ):
                c = 2 * i + par

                @pl.when(i > 0)
                def _():
                    write_desc(c - 2, wbuf, wsem).wait()

                gather_desc(c, gbuf, gsem).wait()
                compute_chunk(c, gbuf, wbuf)
                write_desc(c, wbuf, wsem).start()

                @pl.when(i < nch // 2 - 1)
                def _():
                    gather_desc(c + 2, gbuf, gsem).start()

            return carry

        lax.fori_loop(0, nch // 2, body2, 0)
        write_desc(nch - 2, w0, ws0).wait()
        write_desc(nch - 1, w1, ws1).wait()

    return fused_k


def kernel(input_ids, word_table, pos_table, type_table, ln_gamma, ln_beta):
    B, L = input_ids.shape
    n_tok = B * L
    ids = input_ids.reshape(n_tok).astype(jnp.int32)
    out = _make_sc_fused(B, L, ch=8)(
        ids, word_table, pos_table, type_table, ln_gamma, ln_beta
    )
    return out.reshape(B, L, HIDDEN)


# speedup vs baseline: 1.3472x; 1.3472x over previous
import functools

import jax
import jax.numpy as jnp
from jax import lax
from jax.experimental import pallas as pl
from jax.experimental.pallas import tpu as pltpu
from jax.experimental.pallas import tpu_sc as plsc

HIDDEN = 1024
EPS = 1e-12

_NC = 2
_NS = 16
_NW = _NC * _NS


def _make_sc_gather(n_tok: int, ch: int):
    n_per_w = n_tok // _NW
    nch = n_per_w // ch
    mesh = plsc.VectorSubcoreMesh(core_axis_name="c", subcore_axis_name="s")

    @functools.partial(
        pl.kernel,
        mesh=mesh,
        out_type=jax.ShapeDtypeStruct((n_tok, HIDDEN), jnp.float32),
        scratch_types=[
            pltpu.VMEM((n_per_w,), jnp.int32),
            pltpu.VMEM((ch, HIDDEN), jnp.float32),
            pltpu.VMEM((ch, HIDDEN), jnp.float32),
            pltpu.SemaphoreType.DMA,
            pltpu.SemaphoreType.DMA,
            pltpu.SemaphoreType.DMA,
        ],
    )
    def gather_k(ids_hbm, table_hbm, out_hbm, ids_v, r0, r1, gsem, w0sem, w1sem):
        wid = lax.axis_index("s") * _NC + lax.axis_index("c")
        base = wid * n_per_w
        pltpu.sync_copy(ids_hbm.at[pl.ds(base, n_per_w)], ids_v)

        bufs = (r0, r1)
        wsems = (w0sem, w1sem)

        def start_gather(c, buf):
            return pltpu.async_copy(
                table_hbm.at[ids_v.at[pl.ds(c * ch, ch)]], buf, gsem
            )

        pending = [None, None]
        g_prev = start_gather(0, r0)
        for c in range(nch):
            b = c % 2
            g_prev.wait()
            if c + 1 < nch:
                nb = (c + 1) % 2
                if pending[nb] is not None:
                    pending[nb].wait()
                    pending[nb] = None
                g_prev = start_gather(c + 1, bufs[nb])
            pending[b] = pltpu.async_copy(
                bufs[b], out_hbm.at[pl.ds(base + c * ch, ch)], wsems[b]
            )
        for b in range(2):
            if pending[b] is not None:
                pending[b].wait()

    return gather_k


def _make_tc_ln(n_tok: int, seq_len: int, batch: int, blk: int):
    pos_blocks = seq_len // blk

    def body(x_ref, pos_ref, type_ref, g_ref, b_ref, o_ref):
        x = x_ref[...] + pos_ref[...] + type_ref[0:1, :]
        mu = jnp.mean(x, axis=-1, keepdims=True)
        xc = x - mu
        var = jnp.mean(xc * xc, axis=-1, keepdims=True)
        rstd = lax.rsqrt(var + EPS)
        o_ref[...] = (xc * rstd) * g_ref[0:1, :] + b_ref[0:1, :]

    return pl.pallas_call(
        body,
        grid=(pos_blocks, batch),
        in_specs=[
            pl.BlockSpec((blk, HIDDEN), lambda p, b: (b * pos_blocks + p, 0)),
            pl.BlockSpec((blk, HIDDEN), lambda p, b: (p, 0)),
            pl.BlockSpec((2, HIDDEN), lambda p, b: (0, 0)),
            pl.BlockSpec((1, HIDDEN), lambda p, b: (0, 0)),
            pl.BlockSpec((1, HIDDEN), lambda p, b: (0, 0)),
        ],
        out_specs=pl.BlockSpec((blk, HIDDEN), lambda p, b: (b * pos_blocks + p, 0)),
        out_shape=jax.ShapeDtypeStruct((n_tok, HIDDEN), jnp.float32),
    )


def _make_sc_fused(batch: int, seq_len: int, ch: int):
    n_tok = batch * seq_len
    pw = seq_len // _NW
    cpb = pw // ch
    nch = batch * cpb
    vpr = HIDDEN // 16
    mesh = plsc.VectorSubcoreMesh(core_axis_name="c", subcore_axis_name="s")

    @functools.partial(
        pl.kernel,
        mesh=mesh,
        out_type=jax.ShapeDtypeStruct((n_tok, HIDDEN), jnp.float32),
        scratch_types=[
            pltpu.VMEM((batch * pw,), jnp.int32),
            pltpu.VMEM((pw, HIDDEN), jnp.float32),
            pltpu.VMEM((HIDDEN,), jnp.float32),
            pltpu.VMEM((HIDDEN,), jnp.float32),
            pltpu.VMEM((HIDDEN,), jnp.float32),
            pltpu.VMEM((ch, HIDDEN), jnp.float32),
            pltpu.VMEM((ch, HIDDEN), jnp.float32),
            pltpu.VMEM((ch, HIDDEN), jnp.float32),
            pltpu.SemaphoreType.DMA,
            pltpu.SemaphoreType.DMA,
            pltpu.SemaphoreType.DMA,
        ],
    )
    def fused_k(ids_hbm, table_hbm, pos_hbm, type_hbm, gamma_hbm, beta_hbm,
                out_hbm, ids_v, pt_v, type_v, gamma_v, beta_v,
                g0, g1, wb, gs0, gs1, wsem):
        wid = lax.axis_index("s") * _NC + lax.axis_index("c")
        pbase = wid * pw
        for b in range(batch):
            pltpu.sync_copy(
                ids_hbm.at[pl.ds(b * seq_len + pbase, pw)],
                ids_v.at[pl.ds(b * pw, pw)],
            )
        pltpu.sync_copy(pos_hbm.at[pl.ds(pbase, pw)], pt_v)
        pltpu.sync_copy(type_hbm.at[0], type_v)
        pltpu.sync_copy(gamma_hbm, gamma_v)
        pltpu.sync_copy(beta_hbm, beta_v)

        def fold_type(r, carry):
            for j in range(vpr):
                js = pl.ds(j * 16, 16)
                pt_v[r, js] = pt_v[r, js] + type_v[js]
            return carry

        lax.fori_loop(0, pw, fold_type, 0)

        def gather_desc(c, gbuf, gsem):
            return pltpu.make_async_copy(
                table_hbm.at[ids_v.at[pl.ds(c * ch, ch)]], gbuf, gsem
            )

        def write_desc(c):
            b = c // cpb
            lc = lax.rem(c, cpb)
            dst = b * seq_len + pbase + lc * ch
            return pltpu.make_async_copy(
                wb, out_hbm.at[pl.ds(dst, ch)], wsem
            )

        def permute(x, idx):
            return lax.gather(
                x,
                idx[:, None],
                lax.GatherDimensionNumbers(
                    offset_dims=(),
                    collapsed_slice_dims=(0,),
                    start_index_map=(0,),
                ),
                slice_sizes=(1,),
                mode=lax.GatherScatterMode.PROMISE_IN_BOUNDS,
            )

        def pass1(c, gbuf):
            lc = lax.rem(c, cpb)

            def row_body(r, carry):
                tv_c, qv_c = carry
                acc = [jnp.zeros((16,), jnp.float32) for _ in range(4)]
                qcc = [jnp.zeros((16,), jnp.float32) for _ in range(4)]
                pr = lc * ch + r
                for j in range(vpr):
                    js = pl.ds(j * 16, 16)
                    x = gbuf[r, js] + pt_v[pr, js]
                    gbuf[r, js] = x
                    acc[j % 4] = acc[j % 4] + x
                    qcc[j % 4] = qcc[j % 4] + x * x
                s = (acc[0] + acc[1]) + (acc[2] + acc[3])
                q = (qcc[0] + qcc[1]) + (qcc[2] + qcc[3])
                for d in (1, 2, 4, 8):
                    idx = lax.iota(jnp.int32, 16) ^ d
                    s = s + permute(s, idx)
                    q = q + permute(q, idx)
                lane_r = lax.iota(jnp.int32, 16) == r
                return (
                    jnp.where(lane_r, s, tv_c),
                    jnp.where(lane_r, q, qv_c),
                )

            zeros = jnp.zeros((16,), jnp.float32)
            return lax.fori_loop(0, ch, row_body, (zeros, zeros))

        def chunk_stats(tv, qv):
            mean_v = tv * (1.0 / HIDDEN)
            var_v = qv * (1.0 / HIDDEN) - mean_v * mean_v
            v = var_v + EPS
            seed = lax.bitcast_convert_type(
                jnp.int32(0x5F3759DF) - lax.shift_right_logical(
                    lax.bitcast_convert_type(v, jnp.int32), 1
                ),
                jnp.float32,
            )
            hv = 0.5 * v
            y = seed
            for _ in range(3):
                y = y * (1.5 - hv * y * y)
            means = []
            rstds = []
            for r in range(ch):
                ridx = jnp.full((16,), r, jnp.int32)
                means.append(permute(mean_v, ridx))
                rstds.append(permute(y, ridx))
            return means, rstds

        def pass2(gbuf, means, rstds):
            def col_body(j, carry):
                js = pl.ds(lax.mul(j, 16), 16)
                g = gamma_v[js]
                bb = beta_v[js]
                for r in range(ch):
                    x = gbuf[r, js]
                    wb[r, js] = (x - means[r]) * rstds[r] * g + bb
                return carry

            lax.fori_loop(0, vpr, col_body, 0)

        gather_desc(0, g0, gs0).start()
        gather_desc(1, g1, gs1).start()

        def body2(i, carry):
            for (gbuf, gsem, par) in ((g0, gs0, 0), (g1, gs1, 1)):
                c = 2 * i + par
                gather_desc(c, gbuf, gsem).wait()
                tv, qv = pass1(c, gbuf)
                means, rstds = chunk_stats(tv, qv)

                @pl.when(c > 0)
                def _():
                    write_desc(c - 1).wait()

                pass2(gbuf, means, rstds)
                write_desc(c).start()

                @pl.when(c < nch - 2)
                def _():
                    gather_desc(c + 2, gbuf, gsem).start()

            return carry

        lax.fori_loop(0, nch // 2, body2, 0)
        write_desc(nch - 1).wait()

    return fused_k


def kernel(input_ids, word_table, pos_table, type_table, ln_gamma, ln_beta):
    B, L = input_ids.shape
    n_tok = B * L
    ids = input_ids.reshape(n_tok).astype(jnp.int32)
    out = _make_sc_fused(B, L, ch=16)(
        ids, word_table, pos_table, type_table, ln_gamma, ln_beta
    )
    return out.reshape(B, L, HIDDEN)


# speedup vs baseline: 3.1764x; 2.3577x over previous
import functools

import jax
import jax.numpy as jnp
from jax import lax
from jax.experimental import pallas as pl
from jax.experimental.pallas import tpu as pltpu
from jax.experimental.pallas import tpu_sc as plsc

HIDDEN = 1024
EPS = 1e-12

_NC = 2
_NS = 16
_NW = _NC * _NS


def _make_sc_gather(n_tok: int, ch: int):
    n_per_w = n_tok // _NW
    nch = n_per_w // ch
    mesh = plsc.VectorSubcoreMesh(core_axis_name="c", subcore_axis_name="s")

    @functools.partial(
        pl.kernel,
        mesh=mesh,
        out_type=jax.ShapeDtypeStruct((n_tok, HIDDEN), jnp.float32),
        scratch_types=[
            pltpu.VMEM((n_per_w,), jnp.int32),
            pltpu.VMEM((ch, HIDDEN), jnp.float32),
            pltpu.VMEM((ch, HIDDEN), jnp.float32),
            pltpu.SemaphoreType.DMA,
            pltpu.SemaphoreType.DMA,
            pltpu.SemaphoreType.DMA,
        ],
    )
    def gather_k(ids_hbm, table_hbm, out_hbm, ids_v, r0, r1, gsem, w0sem, w1sem):
        wid = lax.axis_index("s") * _NC + lax.axis_index("c")
        base = wid * n_per_w
        pltpu.sync_copy(ids_hbm.at[pl.ds(base, n_per_w)], ids_v)

        bufs = (r0, r1)
        wsems = (w0sem, w1sem)

        def start_gather(c, buf):
            return pltpu.async_copy(
                table_hbm.at[ids_v.at[pl.ds(c * ch, ch)]], buf, gsem
            )

        pending = [None, None]
        g_prev = start_gather(0, r0)
        for c in range(nch):
            b = c % 2
            g_prev.wait()
            if c + 1 < nch:
                nb = (c + 1) % 2
                if pending[nb] is not None:
                    pending[nb].wait()
                    pending[nb] = None
                g_prev = start_gather(c + 1, bufs[nb])
            pending[b] = pltpu.async_copy(
                bufs[b], out_hbm.at[pl.ds(base + c * ch, ch)], wsems[b]
            )
        for b in range(2):
            if pending[b] is not None:
                pending[b].wait()

    return gather_k


def _make_tc_ln(n_tok: int, seq_len: int, batch: int, blk: int):
    pos_blocks = seq_len // blk

    def body(x_ref, pos_ref, type_ref, g_ref, b_ref, o_ref):
        x = x_ref[...] + pos_ref[...] + type_ref[0:1, :]
        mu = jnp.mean(x, axis=-1, keepdims=True)
        xc = x - mu
        var = jnp.mean(xc * xc, axis=-1, keepdims=True)
        rstd = lax.rsqrt(var + EPS)
        o_ref[...] = (xc * rstd) * g_ref[0:1, :] + b_ref[0:1, :]

    return pl.pallas_call(
        body,
        grid=(pos_blocks, batch),
        in_specs=[
            pl.BlockSpec((blk, HIDDEN), lambda p, b: (b * pos_blocks + p, 0)),
            pl.BlockSpec((blk, HIDDEN), lambda p, b: (p, 0)),
            pl.BlockSpec((2, HIDDEN), lambda p, b: (0, 0)),
            pl.BlockSpec((1, HIDDEN), lambda p, b: (0, 0)),
            pl.BlockSpec((1, HIDDEN), lambda p, b: (0, 0)),
        ],
        out_specs=pl.BlockSpec((blk, HIDDEN), lambda p, b: (b * pos_blocks + p, 0)),
        out_shape=jax.ShapeDtypeStruct((n_tok, HIDDEN), jnp.float32),
    )


def kernel(input_ids, word_table, pos_table, type_table, ln_gamma, ln_beta):
    B, L = input_ids.shape
    n_tok = B * L
    ids = input_ids.reshape(n_tok).astype(jnp.int32)
    rows = _make_sc_gather(n_tok, ch=32)(ids, word_table)
    out = _make_tc_ln(n_tok, L, B, blk=2048)(
        rows,
        pos_table,
        type_table,
        ln_gamma.reshape(1, HIDDEN),
        ln_beta.reshape(1, HIDDEN),
    )
    return out.reshape(B, L, HIDDEN)


# speedup vs baseline: 3.2161x; 1.0125x over previous
import functools

import jax
import jax.numpy as jnp
from jax import lax
from jax.experimental import pallas as pl
from jax.experimental.pallas import tpu as pltpu
from jax.experimental.pallas import tpu_sc as plsc

HIDDEN = 1024
EPS = 1e-12

_NC = 2
_NS = 16
_NW = _NC * _NS


def _make_sc_gather(n_tok: int, ch: int):
    n_per_w = n_tok // _NW
    nch = n_per_w // ch
    mesh = plsc.VectorSubcoreMesh(core_axis_name="c", subcore_axis_name="s")

    @functools.partial(
        pl.kernel,
        mesh=mesh,
        out_type=jax.ShapeDtypeStruct((n_tok, HIDDEN), jnp.float32),
        scratch_types=[
            pltpu.VMEM((n_per_w,), jnp.int32),
            pltpu.VMEM((ch, HIDDEN), jnp.float32),
            pltpu.SemaphoreType.DMA,
        ],
    )
    def gather_k(ids_hbm, table_hbm, out_hbm, ids_v, rows_v, sem):
        wid = lax.axis_index("s") * _NC + lax.axis_index("c")
        base = wid * n_per_w
        pltpu.sync_copy(ids_hbm.at[pl.ds(base, n_per_w)], ids_v)

        def body(c, carry):
            off = pl.multiple_of(c * ch, 8)
            pltpu.async_copy(
                table_hbm.at[ids_v.at[pl.ds(off, ch)]], rows_v, sem
            ).wait()
            pltpu.sync_copy(rows_v, out_hbm.at[pl.ds(base + off, ch)])
            return carry

        lax.fori_loop(0, nch, body, 0)

    return gather_k


def _make_tc_ln(n_tok: int, seq_len: int, batch: int, blk: int):
    pos_blocks = seq_len // blk

    def body(x_ref, pos_ref, type_ref, g_ref, b_ref, o_ref):
        x = x_ref[...] + pos_ref[...] + type_ref[0:1, :]
        mu = jnp.mean(x, axis=-1, keepdims=True)
        xc = x - mu
        var = jnp.mean(xc * xc, axis=-1, keepdims=True)
        rstd = lax.rsqrt(var + EPS)
        o_ref[...] = (xc * rstd) * g_ref[0:1, :] + b_ref[0:1, :]

    return pl.pallas_call(
        body,
        grid=(pos_blocks, batch),
        in_specs=[
            pl.BlockSpec((blk, HIDDEN), lambda p, b: (b * pos_blocks + p, 0)),
            pl.BlockSpec((blk, HIDDEN), lambda p, b: (p, 0)),
            pl.BlockSpec((2, HIDDEN), lambda p, b: (0, 0)),
            pl.BlockSpec((1, HIDDEN), lambda p, b: (0, 0)),
            pl.BlockSpec((1, HIDDEN), lambda p, b: (0, 0)),
        ],
        out_specs=pl.BlockSpec((blk, HIDDEN), lambda p, b: (b * pos_blocks + p, 0)),
        out_shape=jax.ShapeDtypeStruct((n_tok, HIDDEN), jnp.float32),
    )


def kernel(input_ids, word_table, pos_table, type_table, ln_gamma, ln_beta):
    B, L = input_ids.shape
    n_tok = B * L
    ids = input_ids.reshape(n_tok).astype(jnp.int32)
    rows = _make_sc_gather(n_tok, ch=64)(ids, word_table)
    out = _make_tc_ln(n_tok, L, B, blk=2048)(
        rows,
        pos_table,
        type_table,
        ln_gamma.reshape(1, HIDDEN),
        ln_beta.reshape(1, HIDDEN),
    )
    return out.reshape(B, L, HIDDEN)
